# baseline (speedup 1.0000x reference)
import functools

import jax
import jax.numpy as jnp
from jax import lax
from jax.experimental import pallas as pl
from jax.experimental.pallas import tpu as pltpu
from jax.experimental.pallas import tpu_sc as plsc

H = 8
DH = 16
D = 128
NC = 2
NS = 16

_DNUMS = lax.GatherDimensionNumbers(
    offset_dims=(), collapsed_slice_dims=(0,), start_index_map=(0,))


def _vgather(vec, idx):
    return lax.gather(vec, idx.reshape(DH, 1), _DNUMS, (1,),
                      mode=lax.GatherScatterMode.PROMISE_IN_BOUNDS)


def _pick(total, cap, mult):
    for cand in range(cap - cap % mult, 0, -mult):
        if total % cand == 0:
            return cand
    raise ValueError((total, cap, mult))


def _prologue_body(x_ref, wt_ref, ac_ref, b_ref, h_ref, aux_ref):
    xb = x_ref[...]
    h = jnp.dot(xb, wt_ref[...], preferred_element_type=jnp.float32)
    h_ref[...] = h
    aux = jnp.dot(h, ac_ref[...], preferred_element_type=jnp.float32)
    col = lax.broadcasted_iota(jnp.int32, aux.shape, 1)
    aux_ref[...] = aux + jnp.where(col < H, b_ref[0], 0.0)


def _prologue(x, wt, acomb, b, block):
    n = x.shape[0]
    grid = (n // block,)
    return pl.pallas_call(
        _prologue_body,
        grid=grid,
        in_specs=[
            pl.BlockSpec((block, x.shape[1]), lambda i: (i, 0)),
            pl.BlockSpec(wt.shape, lambda i: (0, 0)),
            pl.BlockSpec(acomb.shape, lambda i: (0, 0)),
            pl.BlockSpec(memory_space=pltpu.SMEM),
        ],
        out_specs=[
            pl.BlockSpec((block, D), lambda i: (i, 0)),
            pl.BlockSpec((block, 2 * DH), lambda i: (i, 0)),
        ],
        out_shape=[
            jax.ShapeDtypeStruct((n, D), jnp.float32),
            jax.ShapeDtypeStruct((n, 2 * DH), jnp.float32),
        ],
    )(x, wt, acomb, b)


def _make_sc_kernel(n, e):
    n2 = n // NC
    nd = n2 + 8
    e_per_s = e // NS
    k = _pick(e_per_s, 128, DH)
    nblk = e_per_s // k
    rpt = (n2 // NS) // 8 * 8
    tail = n2 - NS * rpt
    rpa = (n // NS) // 8 * 8
    taila = n - NS * rpa
    mesh = plsc.VectorSubcoreMesh(core_axis_name="c", subcore_axis_name="s")

    @functools.partial(
        pl.kernel,
        out_type=[
            jax.ShapeDtypeStruct((NC, n2, D), jnp.float32),
            jax.ShapeDtypeStruct((NC, n2, DH), jnp.float32),
        ],
        mesh=mesh,
        compiler_params=pltpu.CompilerParams(use_tc_tiling_on_sc=False),
        scratch_types=[
            pltpu.VMEM((k,), jnp.int32),
            pltpu.VMEM((k,), jnp.int32),
            pltpu.VMEM((k,), jnp.int32),
            pltpu.VMEM((k, D), jnp.float32),
            pltpu.VMEM((k, DH), jnp.float32),
            pltpu.VMEM((k, DH), jnp.float32),
            pltpu.VMEM((k, DH), jnp.float32),
            pltpu.VMEM_SHARED((nd, D), jnp.float32),
            pltpu.VMEM_SHARED((nd, DH), jnp.float32),
            pltpu.VMEM_SHARED((n, DH), jnp.float32),
            pltpu.SemaphoreType.DMA,
            pltpu.SemaphoreType.DMA,
            pltpu.SemaphoreType.DMA,
        ],
    )
    def sc_edges(hmat_hbm, an_hbm, src_hbm, dst_hbm, z128_hbm, z16_hbm,
                 num_out, den_out,
                 srcv, dstv, dstg,
                 hrows, asrc, adst, wbuf, num_s, den_s, an_s,
                 sem1, sem2, sem3):
        c = lax.axis_index("c")
        s = lax.axis_index("s")
        nbase = c * n2

        r0 = s * rpt
        pltpu.sync_copy(z128_hbm.at[pl.ds(r0, rpt)], num_s.at[pl.ds(r0, rpt)])
        a0 = s * rpa
        pltpu.sync_copy(an_hbm.at[pl.ds(a0, rpa)], an_s.at[pl.ds(a0, rpa)])
        pltpu.sync_copy(z16_hbm.at[pl.ds(r0, rpt)], den_s.at[pl.ds(r0, rpt)])
        zt0 = NS * rpt
        zcnt = nd - zt0

        @pl.when(s == 0)
        def _():
            pltpu.sync_copy(z128_hbm.at[pl.ds(zt0, zcnt)],
                            num_s.at[pl.ds(zt0, zcnt)])
            pltpu.sync_copy(z16_hbm.at[pl.ds(zt0, zcnt)],
                            den_s.at[pl.ds(zt0, zcnt)])
        if taila:
            @pl.when(s == 1)
            def _():
                t0 = NS * rpa
                pltpu.sync_copy(an_hbm.at[pl.ds(t0, taila)],
                                an_s.at[pl.ds(t0, taila)])

        plsc.subcore_barrier()

        ebase = s * e_per_s
        lane = lax.iota(jnp.int32, DH)
        rot = (lane + H) % DH
        dump8 = n2 + (lane & (H - 1))

        def blk_body(i, carry):
            base = ebase + i * k
            pltpu.sync_copy(src_hbm.at[pl.ds(base, k)], srcv)
            pltpu.sync_copy(dst_hbm.at[pl.ds(base, k)], dstg)
            for q in range(k // DH):
                dv = dstg[pl.ds(q * DH, DH)]
                dvl = dv - nbase
                ki = (1 + (dvl >> 31)) * (1 + ((n2 - 1 - dvl) >> 31))
                dstv[pl.ds(q * DH, DH)] = dvl * ki + dump8 * (1 - ki)
            cp1 = pltpu.async_copy(hmat_hbm.at[srcv], hrows, sem1)
            cp2 = pltpu.async_copy(an_s.at[srcv], asrc, sem2)
            cp3 = pltpu.async_copy(an_s.at[dstg], adst, sem3)
            cp2.wait()
            cp3.wait()

            cp1.wait()

            def chunk_body(ch, carry2):
                j0 = ch * DH
                dvchunk = dstv[pl.ds(j0, DH)]
                for jj in range(DH):
                    j = j0 + jj

                    @pl.when(dvchunk[jj] < n2)
                    def _():
                        t = asrc[j, :] + _vgather(adst[j, :], rot)
                        t = jnp.maximum(t, 0.2 * t)
                        w = jnp.exp(t)
                        wbuf[j, :] = w
                        for h in range(H):
                            hrows[j, pl.ds(h * DH, DH)] = (
                                hrows[j, pl.ds(h * DH, DH)] * w[h])
                return carry2

            lax.fori_loop(0, k // DH, chunk_body, 0)
            pltpu.sync_copy(wbuf, den_s.at[dstv], add=True)
            pltpu.sync_copy(hrows, num_s.at[dstv], add=True)
            return carry

        lax.fori_loop(0, nblk, blk_body, 0)

        plsc.subcore_barrier()
        pltpu.sync_copy(num_s.at[pl.ds(r0, rpt)],
                        num_out.at[c, pl.ds(r0, rpt)])
        pltpu.sync_copy(den_s.at[pl.ds(r0, rpt)],
                        den_out.at[c, pl.ds(r0, rpt)])
        if tail:
            @pl.when(s == 0)
            def _():
                t0 = NS * rpt
                pltpu.sync_copy(num_s.at[pl.ds(t0, tail)],
                                num_out.at[c, pl.ds(t0, tail)])
                pltpu.sync_copy(den_s.at[pl.ds(t0, tail)],
                                den_out.at[c, pl.ds(t0, tail)])

    return sc_edges


def _epilogue_body(num_ref, d_ref, h_ref, aip_ref, ajp_ref, s16_ref, o_ref):
    num = num_ref[0]
    d16 = d_ref[0]
    t = aip_ref[...] + ajp_ref[...]
    t = jnp.maximum(t, 0.2 * t)
    w16 = jnp.exp(t)
    s16 = s16_ref[...]
    wfull = jnp.dot(w16, s16, preferred_element_type=jnp.float32)
    dfull = jnp.dot(d16 + w16, s16, preferred_element_type=jnp.float32)
    o_ref[...] = (num + wfull * h_ref[...]) / dfull


def _epilogue(num2, den, hmat, aip, ajp, s16, block):
    n = hmat.shape[0]
    n2 = num2.shape[1]
    bpc = n2 // block
    grid = (n // block,)
    return pl.pallas_call(
        _epilogue_body,
        grid=grid,
        in_specs=[
            pl.BlockSpec((1, block, D), lambda i: (i // bpc, i % bpc, 0)),
            pl.BlockSpec((1, block, DH), lambda i: (i // bpc, i % bpc, 0)),
            pl.BlockSpec((block, D), lambda i: (i, 0)),
            pl.BlockSpec((block, DH), lambda i: (i, 0)),
            pl.BlockSpec((block, DH), lambda i: (i, 0)),
            pl.BlockSpec((DH, D), lambda i: (0, 0)),
        ],
        out_specs=pl.BlockSpec((block, D), lambda i: (i, 0)),
        out_shape=jax.ShapeDtypeStruct((n, D), jnp.float32),
    )(num2, den, hmat, aip, ajp, s16)


def kernel(x, edge_index, W, att_w, att_b):
    n = x.shape[0]
    e = edge_index.shape[1]
    assert W.shape == (D, x.shape[1]) and att_w.shape == (1, 2 * DH)
    assert e % NS == 0 and n % (2 * 8) == 0

    wt = W.T
    a1 = att_w[0, :DH]
    a2 = att_w[0, DH:]
    eye = jnp.eye(H, dtype=jnp.float32)
    pad8 = jnp.zeros((D, H), dtype=jnp.float32)
    A1 = jnp.concatenate([jnp.kron(eye, a1.reshape(DH, 1)), pad8], axis=1)
    A2 = jnp.concatenate([jnp.kron(eye, a2.reshape(DH, 1)), pad8], axis=1)
    acomb = jnp.concatenate([A1, A2], axis=1)
    s16 = jnp.concatenate(
        [jnp.kron(eye, jnp.ones((1, DH), dtype=jnp.float32)),
         jnp.zeros((H, D), dtype=jnp.float32)], axis=0)

    hmat, aux = _prologue(x, wt, acomb, att_b.astype(jnp.float32), block=1000)
    aip = aux[:, :DH]
    ajp = aux[:, DH:]
    anode = jnp.concatenate([aux[:, :H], aux[:, DH:DH + H]], axis=1)

    src = edge_index[0]
    dst = edge_index[1]
    n2 = n // NC
    z128 = jnp.zeros((n2 + 8, D), jnp.float32)
    z16 = jnp.zeros((n2 + 8, DH), jnp.float32)

    sc_fn = _make_sc_kernel(n, e)
    num2, den = sc_fn(hmat, anode, src, dst, z128, z16)

    return _epilogue(num2, den, hmat, aip, ajp, s16, block=1000)


# speedup vs baseline: 1.4214x; 1.4214x over previous
import functools

import jax
import jax.numpy as jnp
from jax import lax
from jax.experimental import pallas as pl
from jax.experimental.pallas import tpu as pltpu
from jax.experimental.pallas import tpu_sc as plsc

H = 8
DH = 16
D = 128
NC = 2
NS = 16

_DNUMS = lax.GatherDimensionNumbers(
    offset_dims=(), collapsed_slice_dims=(0,), start_index_map=(0,))


def _vgather(vec, idx):
    return lax.gather(vec, idx.reshape(DH, 1), _DNUMS, (1,),
                      mode=lax.GatherScatterMode.PROMISE_IN_BOUNDS)


def _pick(total, cap, mult):
    for cand in range(cap - cap % mult, 0, -mult):
        if total % cand == 0:
            return cand
    raise ValueError((total, cap, mult))


def _prologue_body(x_ref, wt_ref, ac_ref, b_ref, h_ref, aux_ref):
    xb = x_ref[...]
    h = jnp.dot(xb, wt_ref[...], preferred_element_type=jnp.float32)
    h_ref[...] = h
    aux = jnp.dot(h, ac_ref[...], preferred_element_type=jnp.float32)
    col = lax.broadcasted_iota(jnp.int32, aux.shape, 1)
    aux_ref[...] = aux + jnp.where(col < H, b_ref[0], 0.0)


def _prologue(x, wt, acomb, b, block):
    n = x.shape[0]
    grid = (n // block,)
    return pl.pallas_call(
        _prologue_body,
        grid=grid,
        in_specs=[
            pl.BlockSpec((block, x.shape[1]), lambda i: (i, 0)),
            pl.BlockSpec(wt.shape, lambda i: (0, 0)),
            pl.BlockSpec(acomb.shape, lambda i: (0, 0)),
            pl.BlockSpec(memory_space=pltpu.SMEM),
        ],
        out_specs=[
            pl.BlockSpec((block, D), lambda i: (i, 0)),
            pl.BlockSpec((block, 2 * DH), lambda i: (i, 0)),
        ],
        out_shape=[
            jax.ShapeDtypeStruct((n, D), jnp.float32),
            jax.ShapeDtypeStruct((n, 2 * DH), jnp.float32),
        ],
    )(x, wt, acomb, b)


def _make_sc_kernel(n, e):
    n2 = n // NC
    nd = n2 + 8
    e_per_s = e // NS
    k = _pick(e_per_s, 128, DH)
    nblk = e_per_s // k
    rpt = (n2 // NS) // 8 * 8
    tail = n2 - NS * rpt
    rpa = (n // NS) // 8 * 8
    taila = n - NS * rpa
    mesh = plsc.VectorSubcoreMesh(core_axis_name="c", subcore_axis_name="s")

    @functools.partial(
        pl.kernel,
        out_type=[
            jax.ShapeDtypeStruct((NC, n2, D), jnp.float32),
            jax.ShapeDtypeStruct((NC, n2, DH), jnp.float32),
        ],
        mesh=mesh,
        compiler_params=pltpu.CompilerParams(use_tc_tiling_on_sc=False),
        scratch_types=(
            [pltpu.VMEM((k,), jnp.int32)] * 6
            + [pltpu.VMEM((k, D), jnp.float32)] * 2
            + [pltpu.VMEM((k, DH), jnp.float32)] * 4
            + [
                pltpu.VMEM((k, DH), jnp.float32),
                pltpu.VMEM_SHARED((nd, D), jnp.float32),
                pltpu.VMEM_SHARED((nd, DH), jnp.float32),
                pltpu.VMEM_SHARED((n, DH), jnp.float32),
            ]
            + [pltpu.SemaphoreType.DMA] * 10
        ),
    )
    def sc_edges(hmat_hbm, an_hbm, src_hbm, dst_hbm, z128_hbm, z16_hbm,
                 num_out, den_out,
                 srcv0, srcv1, dstv0, dstv1, dstg0, dstg1,
                 hrows0, hrows1, asrc0, asrc1, adst0, adst1,
                 wbuf, num_s, den_s, an_s,
                 si0, sd0, sh0, sa0, sb0, si1, sd1, sh1, sa1, sb1):
        c = lax.axis_index("c")
        s = lax.axis_index("s")
        nbase = c * n2

        r0 = s * rpt
        pltpu.sync_copy(z128_hbm.at[pl.ds(r0, rpt)], num_s.at[pl.ds(r0, rpt)])
        a0 = s * rpa
        pltpu.sync_copy(an_hbm.at[pl.ds(a0, rpa)], an_s.at[pl.ds(a0, rpa)])
        pltpu.sync_copy(z16_hbm.at[pl.ds(r0, rpt)], den_s.at[pl.ds(r0, rpt)])
        zt0 = NS * rpt
        zcnt = nd - zt0

        @pl.when(s == 0)
        def _():
            pltpu.sync_copy(z128_hbm.at[pl.ds(zt0, zcnt)],
                            num_s.at[pl.ds(zt0, zcnt)])
            pltpu.sync_copy(z16_hbm.at[pl.ds(zt0, zcnt)],
                            den_s.at[pl.ds(zt0, zcnt)])
        if taila:
            @pl.when(s == 1)
            def _():
                t0 = NS * rpa
                pltpu.sync_copy(an_hbm.at[pl.ds(t0, taila)],
                                an_s.at[pl.ds(t0, taila)])

        plsc.subcore_barrier()

        ebase = s * e_per_s
        lane = lax.iota(jnp.int32, DH)
        rot = (lane + H) % DH
        dump8 = n2 + (lane & (H - 1))

        slot0 = (srcv0, dstg0, dstv0, hrows0, asrc0, adst0,
                 si0, sd0, sh0, sa0, sb0)
        slot1 = (srcv1, dstg1, dstv1, hrows1, asrc1, adst1,
                 si1, sd1, sh1, sa1, sb1)

        def issue_idx(b, slot):
            srcv_, dstg_ = slot[0], slot[1]
            si, sd = slot[6], slot[7]
            base = ebase + b * k
            pltpu.async_copy(src_hbm.at[pl.ds(base, k)], srcv_, si)
            pltpu.async_copy(dst_hbm.at[pl.ds(base, k)], dstg_, sd)

        def wait_idx(slot):
            srcv_, dstg_ = slot[0], slot[1]
            si, sd = slot[6], slot[7]
            pltpu.make_async_copy(src_hbm.at[pl.ds(0, k)], srcv_, si).wait()
            pltpu.make_async_copy(dst_hbm.at[pl.ds(0, k)], dstg_, sd).wait()

        def localize(slot):
            dstg_, dstv_ = slot[1], slot[2]
            for q in range(k // DH):
                dv = dstg_[pl.ds(q * DH, DH)]
                dvl = dv - nbase
                ki = (1 + (dvl >> 31)) * (1 + ((n2 - 1 - dvl) >> 31))
                dstv_[pl.ds(q * DH, DH)] = dvl * ki + dump8 * (1 - ki)

        def issue_gather(slot):
            srcv_, dstg_, hrows_, asrc_, adst_ = (slot[0], slot[1], slot[3],
                                                  slot[4], slot[5])
            sh, sa, sb = slot[8], slot[9], slot[10]
            pltpu.async_copy(hmat_hbm.at[srcv_], hrows_, sh)
            pltpu.async_copy(an_s.at[srcv_], asrc_, sa)
            pltpu.async_copy(an_s.at[dstg_], adst_, sb)

        def wait_gather(slot):
            hrows_, asrc_, adst_ = slot[3], slot[4], slot[5]
            sh, sa, sb = slot[8], slot[9], slot[10]
            pltpu.make_async_copy(hmat_hbm.at[pl.ds(0, k)], hrows_, sh).wait()
            pltpu.make_async_copy(an_s.at[pl.ds(0, k)], asrc_, sa).wait()
            pltpu.make_async_copy(an_s.at[pl.ds(0, k)], adst_, sb).wait()

        def process(slot):
            dstv_, hrows_, asrc_, adst_ = slot[2], slot[3], slot[4], slot[5]

            def score_body(j, carry2):
                t = asrc_[j, :] + _vgather(adst_[j, :], rot)
                t = jnp.maximum(t, 0.2 * t)
                wbuf[j, :] = jnp.exp(t)
                return carry2

            lax.fori_loop(0, k, score_body, 0, unroll=4)
            pltpu.sync_copy(wbuf, den_s.at[dstv_], add=True)

            def scale_body(j, carry2):
                wv = wbuf[j, :]
                for h in range(H):
                    hrows_[j, pl.ds(h * DH, DH)] = (
                        hrows_[j, pl.ds(h * DH, DH)] * wv[h])
                return carry2

            lax.fori_loop(0, k, scale_body, 0, unroll=2)
            pltpu.sync_copy(hrows_, num_s.at[dstv_], add=True)

        assert nblk % 2 == 0 and nblk >= 4
        last = nblk - 1
        issue_idx(0, slot0)
        wait_idx(slot0)
        localize(slot0)
        issue_gather(slot0)
        issue_idx(1, slot1)

        def pair_body(g, carry):
            b0 = 2 * g
            wait_idx(slot1)
            localize(slot1)
            issue_gather(slot1)
            wait_gather(slot0)
            process(slot0)
            issue_idx(jnp.minimum(b0 + 2, last), slot0)
            wait_idx(slot0)
            localize(slot0)
            issue_gather(slot0)
            wait_gather(slot1)
            process(slot1)
            issue_idx(jnp.minimum(b0 + 3, last), slot1)
            return carry

        lax.fori_loop(0, nblk // 2, pair_body, 0)
        wait_idx(slot1)
        wait_gather(slot0)

        plsc.subcore_barrier()
        pltpu.sync_copy(num_s.at[pl.ds(r0, rpt)],
                        num_out.at[c, pl.ds(r0, rpt)])
        pltpu.sync_copy(den_s.at[pl.ds(r0, rpt)],
                        den_out.at[c, pl.ds(r0, rpt)])
        if tail:
            @pl.when(s == 0)
            def _():
                t0 = NS * rpt
                pltpu.sync_copy(num_s.at[pl.ds(t0, tail)],
                                num_out.at[c, pl.ds(t0, tail)])
                pltpu.sync_copy(den_s.at[pl.ds(t0, tail)],
                                den_out.at[c, pl.ds(t0, tail)])

    return sc_edges


def _epilogue_body(num_ref, d_ref, h_ref, aip_ref, ajp_ref, s16_ref, o_ref):
    num = num_ref[0]
    d16 = d_ref[0]
    t = aip_ref[...] + ajp_ref[...]
    t = jnp.maximum(t, 0.2 * t)
    w16 = jnp.exp(t)
    s16 = s16_ref[...]
    wfull = jnp.dot(w16, s16, preferred_element_type=jnp.float32)
    dfull = jnp.dot(d16 + w16, s16, preferred_element_type=jnp.float32)
    o_ref[...] = (num + wfull * h_ref[...]) / dfull


def _epilogue(num2, den, hmat, aip, ajp, s16, block):
    n = hmat.shape[0]
    n2 = num2.shape[1]
    bpc = n2 // block
    grid = (n // block,)
    return pl.pallas_call(
        _epilogue_body,
        grid=grid,
        in_specs=[
            pl.BlockSpec((1, block, D), lambda i: (i // bpc, i % bpc, 0)),
            pl.BlockSpec((1, block, DH), lambda i: (i // bpc, i % bpc, 0)),
            pl.BlockSpec((block, D), lambda i: (i, 0)),
            pl.BlockSpec((block, DH), lambda i: (i, 0)),
            pl.BlockSpec((block, DH), lambda i: (i, 0)),
            pl.BlockSpec((DH, D), lambda i: (0, 0)),
        ],
        out_specs=pl.BlockSpec((block, D), lambda i: (i, 0)),
        out_shape=jax.ShapeDtypeStruct((n, D), jnp.float32),
    )(num2, den, hmat, aip, ajp, s16)


def kernel(x, edge_index, W, att_w, att_b):
    n = x.shape[0]
    e = edge_index.shape[1]
    assert W.shape == (D, x.shape[1]) and att_w.shape == (1, 2 * DH)
    assert e % NS == 0 and n % (2 * 8) == 0

    wt = W.T
    a1 = att_w[0, :DH]
    a2 = att_w[0, DH:]
    eye = jnp.eye(H, dtype=jnp.float32)
    pad8 = jnp.zeros((D, H), dtype=jnp.float32)
    A1 = jnp.concatenate([jnp.kron(eye, a1.reshape(DH, 1)), pad8], axis=1)
    A2 = jnp.concatenate([jnp.kron(eye, a2.reshape(DH, 1)), pad8], axis=1)
    acomb = jnp.concatenate([A1, A2], axis=1)
    s16 = jnp.concatenate(
        [jnp.kron(eye, jnp.ones((1, DH), dtype=jnp.float32)),
         jnp.zeros((H, D), dtype=jnp.float32)], axis=0)

    hmat, aux = _prologue(x, wt, acomb, att_b.astype(jnp.float32), block=1000)
    aip = aux[:, :DH]
    ajp = aux[:, DH:]
    anode = jnp.concatenate([aux[:, :H], aux[:, DH:DH + H]], axis=1)

    src = edge_index[0]
    dst = edge_index[1]
    n2 = n // NC
    z128 = jnp.zeros((n2 + 8, D), jnp.float32)
    z16 = jnp.zeros((n2 + 8, DH), jnp.float32)

    sc_fn = _make_sc_kernel(n, e)
    num2, den = sc_fn(hmat, anode, src, dst, z128, z16)

    return _epilogue(num2, den, hmat, aip, ajp, s16, block=1000)


# speedup vs baseline: 3.0850x; 2.1703x over previous
import functools

import jax
import jax.numpy as jnp
from jax import lax
from jax.experimental import pallas as pl
from jax.experimental.pallas import tpu as pltpu
from jax.experimental.pallas import tpu_sc as plsc

H = 8
DH = 16
D = 128
NC = 2
NS = 16

_DNUMS = lax.GatherDimensionNumbers(
    offset_dims=(), collapsed_slice_dims=(0,), start_index_map=(0,))


def _vgather(vec, idx):
    return lax.gather(vec, idx.reshape(DH, 1), _DNUMS, (1,),
                      mode=lax.GatherScatterMode.PROMISE_IN_BOUNDS)


def _pick(total, cap, mult):
    for cand in range(cap - cap % mult, 0, -mult):
        if total % cand == 0:
            return cand
    raise ValueError((total, cap, mult))


def _prologue_body(x_ref, wt_ref, ac_ref, b_ref, h_ref, aux_ref):
    xb = x_ref[...]
    h = jnp.dot(xb, wt_ref[...], preferred_element_type=jnp.float32)
    h_ref[...] = h
    aux = jnp.dot(h, ac_ref[...], preferred_element_type=jnp.float32)
    col = lax.broadcasted_iota(jnp.int32, aux.shape, 1)
    aux_ref[...] = aux + jnp.where(col < H, b_ref[0], 0.0)


def _prologue(x, wt, acomb, b, block):
    n = x.shape[0]
    grid = (n // block,)
    return pl.pallas_call(
        _prologue_body,
        grid=grid,
        in_specs=[
            pl.BlockSpec((block, x.shape[1]), lambda i: (i, 0)),
            pl.BlockSpec(wt.shape, lambda i: (0, 0)),
            pl.BlockSpec(acomb.shape, lambda i: (0, 0)),
            pl.BlockSpec(memory_space=pltpu.SMEM),
        ],
        out_specs=[
            pl.BlockSpec((block, D), lambda i: (i, 0)),
            pl.BlockSpec((block, 2 * DH), lambda i: (i, 0)),
        ],
        out_shape=[
            jax.ShapeDtypeStruct((n, D), jnp.float32),
            jax.ShapeDtypeStruct((n, 2 * DH), jnp.float32),
        ],
    )(x, wt, acomb, b)


def _make_sc_kernel(n, e):
    n2 = n // NC
    nd = n2 + 8
    e_per_s = e // NS
    k = _pick(e_per_s, 128, DH)
    nblk = e_per_s // k
    rpt = (n2 // NS) // 8 * 8
    tail = n2 - NS * rpt
    rpa = (n // NS) // 8 * 8
    taila = n - NS * rpa
    mesh = plsc.VectorSubcoreMesh(core_axis_name="c", subcore_axis_name="s")

    @functools.partial(
        pl.kernel,
        out_type=[
            jax.ShapeDtypeStruct((NC, n2, D), jnp.float32),
            jax.ShapeDtypeStruct((NC, n2, DH), jnp.float32),
        ],
        mesh=mesh,
        compiler_params=pltpu.CompilerParams(use_tc_tiling_on_sc=False),
        scratch_types=(
            [pltpu.VMEM((k,), jnp.int32)] * 6
            + [pltpu.VMEM((k, D), jnp.float32)] * 2
            + [pltpu.VMEM((k, DH), jnp.float32)] * 4
            + [
                pltpu.VMEM((k, DH), jnp.float32),
                pltpu.VMEM_SHARED((nd, D), jnp.float32),
                pltpu.VMEM_SHARED((nd, DH), jnp.float32),
                pltpu.VMEM_SHARED((n, DH), jnp.float32),
            ]
            + [pltpu.SemaphoreType.DMA] * 10
        ),
    )
    def sc_edges(hmat_hbm, an_hbm, src_hbm, dst_hbm, z128_hbm, z16_hbm,
                 num_out, den_out,
                 srcv0, srcv1, dstv0, dstv1, dstg0, dstg1,
                 hrows0, hrows1, asrc0, asrc1, adst0, adst1,
                 wbuf, num_s, den_s, an_s,
                 si0, sd0, sh0, sa0, sb0, si1, sd1, sh1, sa1, sb1):
        c = lax.axis_index("c")
        s = lax.axis_index("s")
        nbase = c * n2

        r0 = s * rpt
        pltpu.sync_copy(z128_hbm.at[pl.ds(r0, rpt)], num_s.at[pl.ds(r0, rpt)])
        a0 = s * rpa
        pltpu.sync_copy(an_hbm.at[pl.ds(a0, rpa)], an_s.at[pl.ds(a0, rpa)])
        pltpu.sync_copy(z16_hbm.at[pl.ds(r0, rpt)], den_s.at[pl.ds(r0, rpt)])
        zt0 = NS * rpt
        zcnt = nd - zt0

        @pl.when(s == 0)
        def _():
            pltpu.sync_copy(z128_hbm.at[pl.ds(zt0, zcnt)],
                            num_s.at[pl.ds(zt0, zcnt)])
            pltpu.sync_copy(z16_hbm.at[pl.ds(zt0, zcnt)],
                            den_s.at[pl.ds(zt0, zcnt)])
        if taila:
            @pl.when(s == 1)
            def _():
                t0 = NS * rpa
                pltpu.sync_copy(an_hbm.at[pl.ds(t0, taila)],
                                an_s.at[pl.ds(t0, taila)])

        plsc.subcore_barrier()

        ebase = s * e_per_s
        lane = lax.iota(jnp.int32, DH)
        rot = (lane + H) % DH
        dump8 = n2 + (lane & (H - 1))

        slot0 = (srcv0, dstg0, dstv0, hrows0, asrc0, adst0,
                 si0, sd0, sh0, sa0, sb0)
        slot1 = (srcv1, dstg1, dstv1, hrows1, asrc1, adst1,
                 si1, sd1, sh1, sa1, sb1)

        def issue_idx(b, slot):
            srcv_, dstg_ = slot[0], slot[1]
            si, sd = slot[6], slot[7]
            base = ebase + b * k
            pltpu.async_copy(src_hbm.at[pl.ds(base, k)], srcv_, si)
            pltpu.async_copy(dst_hbm.at[pl.ds(base, k)], dstg_, sd)

        def wait_idx(slot):
            srcv_, dstg_ = slot[0], slot[1]
            si, sd = slot[6], slot[7]
            pltpu.make_async_copy(src_hbm.at[pl.ds(0, k)], srcv_, si).wait()
            pltpu.make_async_copy(dst_hbm.at[pl.ds(0, k)], dstg_, sd).wait()

        def localize(slot):
            dstg_, dstv_ = slot[1], slot[2]
            for q in range(k // DH):
                dv = dstg_[pl.ds(q * DH, DH)]
                dvl = dv - nbase
                ki = (1 + (dvl >> 31)) * (1 + ((n2 - 1 - dvl) >> 31))
                dstv_[pl.ds(q * DH, DH)] = dvl * ki + dump8 * (1 - ki)

        def issue_gather(slot):
            srcv_, dstg_, hrows_, asrc_, adst_ = (slot[0], slot[1], slot[3],
                                                  slot[4], slot[5])
            sh, sa, sb = slot[8], slot[9], slot[10]
            pltpu.async_copy(hmat_hbm.at[srcv_], hrows_, sh)
            pltpu.async_copy(an_s.at[srcv_], asrc_, sa)
            pltpu.async_copy(an_s.at[dstg_], adst_, sb)

        def wait_gather(slot):
            hrows_, asrc_, adst_ = slot[3], slot[4], slot[5]
            sh, sa, sb = slot[8], slot[9], slot[10]
            pltpu.make_async_copy(hmat_hbm.at[pl.ds(0, k)], hrows_, sh).wait()
            pltpu.make_async_copy(an_s.at[pl.ds(0, k)], asrc_, sa).wait()
            pltpu.make_async_copy(an_s.at[pl.ds(0, k)], adst_, sb).wait()

        def process(slot):
            dstv_, hrows_, asrc_, adst_ = slot[2], slot[3], slot[4], slot[5]

            @functools.partial(plsc.parallel_loop, 0, k, unroll=8)
            def _(j):
                t = asrc_[j, :] + _vgather(adst_[j, :], rot)
                t = jnp.maximum(t, 0.2 * t)
                wbuf[j, :] = jnp.exp(t)

            pltpu.sync_copy(wbuf, den_s.at[dstv_], add=True)

            @functools.partial(plsc.parallel_loop, 0, k, unroll=4)
            def _(j):
                wv = wbuf[j, :]
                for h in range(H):
                    hrows_[j, pl.ds(h * DH, DH)] = (
                        hrows_[j, pl.ds(h * DH, DH)] * wv[h])

            pltpu.sync_copy(hrows_, num_s.at[dstv_], add=True)

        assert nblk % 2 == 0 and nblk >= 4
        last = nblk - 1
        issue_idx(0, slot0)
        wait_idx(slot0)
        localize(slot0)
        issue_gather(slot0)
        issue_idx(1, slot1)

        def pair_body(g, carry):
            b0 = 2 * g
            wait_idx(slot1)
            localize(slot1)
            issue_gather(slot1)
            wait_gather(slot0)
            process(slot0)
            issue_idx(jnp.minimum(b0 + 2, last), slot0)
            wait_idx(slot0)
            localize(slot0)
            issue_gather(slot0)
            wait_gather(slot1)
            process(slot1)
            issue_idx(jnp.minimum(b0 + 3, last), slot1)
            return carry

        lax.fori_loop(0, nblk // 2, pair_body, 0)
        wait_idx(slot1)
        wait_gather(slot0)

        plsc.subcore_barrier()
        pltpu.sync_copy(num_s.at[pl.ds(r0, rpt)],
                        num_out.at[c, pl.ds(r0, rpt)])
        pltpu.sync_copy(den_s.at[pl.ds(r0, rpt)],
                        den_out.at[c, pl.ds(r0, rpt)])
        if tail:
            @pl.when(s == 0)
            def _():
                t0 = NS * rpt
                pltpu.sync_copy(num_s.at[pl.ds(t0, tail)],
                                num_out.at[c, pl.ds(t0, tail)])
                pltpu.sync_copy(den_s.at[pl.ds(t0, tail)],
                                den_out.at[c, pl.ds(t0, tail)])

    return sc_edges


def _epilogue_body(num_ref, d_ref, h_ref, aip_ref, ajp_ref, s16_ref, o_ref):
    num = num_ref[0]
    d16 = d_ref[0]
    t = aip_ref[...] + ajp_ref[...]
    t = jnp.maximum(t, 0.2 * t)
    w16 = jnp.exp(t)
    s16 = s16_ref[...]
    wfull = jnp.dot(w16, s16, preferred_element_type=jnp.float32)
    dfull = jnp.dot(d16 + w16, s16, preferred_element_type=jnp.float32)
    o_ref[...] = (num + wfull * h_ref[...]) / dfull


def _epilogue(num2, den, hmat, aip, ajp, s16, block):
    n = hmat.shape[0]
    n2 = num2.shape[1]
    bpc = n2 // block
    grid = (n // block,)
    return pl.pallas_call(
        _epilogue_body,
        grid=grid,
        in_specs=[
            pl.BlockSpec((1, block, D), lambda i: (i // bpc, i % bpc, 0)),
            pl.BlockSpec((1, block, DH), lambda i: (i // bpc, i % bpc, 0)),
            pl.BlockSpec((block, D), lambda i: (i, 0)),
            pl.BlockSpec((block, DH), lambda i: (i, 0)),
            pl.BlockSpec((block, DH), lambda i: (i, 0)),
            pl.BlockSpec((DH, D), lambda i: (0, 0)),
        ],
        out_specs=pl.BlockSpec((block, D), lambda i: (i, 0)),
        out_shape=jax.ShapeDtypeStruct((n, D), jnp.float32),
    )(num2, den, hmat, aip, ajp, s16)


def kernel(x, edge_index, W, att_w, att_b):
    n = x.shape[0]
    e = edge_index.shape[1]
    assert W.shape == (D, x.shape[1]) and att_w.shape == (1, 2 * DH)
    assert e % NS == 0 and n % (2 * 8) == 0

    wt = W.T
    a1 = att_w[0, :DH]
    a2 = att_w[0, DH:]
    eye = jnp.eye(H, dtype=jnp.float32)
    pad8 = jnp.zeros((D, H), dtype=jnp.float32)
    A1 = jnp.concatenate([jnp.kron(eye, a1.reshape(DH, 1)), pad8], axis=1)
    A2 = jnp.concatenate([jnp.kron(eye, a2.reshape(DH, 1)), pad8], axis=1)
    acomb = jnp.concatenate([A1, A2], axis=1)
    s16 = jnp.concatenate(
        [jnp.kron(eye, jnp.ones((1, DH), dtype=jnp.float32)),
         jnp.zeros((H, D), dtype=jnp.float32)], axis=0)

    hmat, aux = _prologue(x, wt, acomb, att_b.astype(jnp.float32), block=1000)
    aip = aux[:, :DH]
    ajp = aux[:, DH:]
    anode = jnp.concatenate([aux[:, :H], aux[:, DH:DH + H]], axis=1)

    src = edge_index[0]
    dst = edge_index[1]
    n2 = n // NC
    z128 = jnp.zeros((n2 + 8, D), jnp.float32)
    z16 = jnp.zeros((n2 + 8, DH), jnp.float32)

    sc_fn = _make_sc_kernel(n, e)
    num2, den = sc_fn(hmat, anode, src, dst, z128, z16)

    return _epilogue(num2, den, hmat, aip, ajp, s16, block=1000)
